# baseline (speedup 1.0000x reference)
import functools

import numpy as np

import jax
import jax.numpy as jnp
from jax.experimental import pallas as pl
from jax.experimental.shard_map import shard_map
from jax.sharding import Mesh, PartitionSpec as P

N = 8192
D = 256
H = 256
C = 64

BM_BIG = 256
BM_SMALL = 1024

_bf16 = jnp.bfloat16
_f32 = jnp.float32


def _neigh_lin_kernel(x_ref, w_ref, o_ref):
    o_ref[...] = jnp.dot(
        x_ref[...].astype(_bf16), w_ref[...], preferred_element_type=_f32
    ).astype(_bf16)


def _agg_deg_kernel(adj_ref, h_ref, agg_ref, deg_ref):
    a = adj_ref[...]
    deg_ref[...] = jnp.sum(a, axis=1, keepdims=True) + 1.0
    agg_ref[...] = jnp.dot(
        a.astype(_bf16), h_ref[...], preferred_element_type=_f32
    )


def _agg_kernel(adj_ref, h_ref, agg_ref):
    agg_ref[...] = jnp.dot(
        adj_ref[...].astype(_bf16), h_ref[...], preferred_element_type=_f32
    )


def _mid_kernel(x_ref, agg_ref, deg_ref, wa_ref, wb_ref, wn_ref,
                h1f_ref, h1_ref):
    hn = (agg_ref[...] / deg_ref[...]).astype(_bf16)
    z = jnp.dot(x_ref[...].astype(_bf16), wa_ref[...],
                preferred_element_type=_f32)
    z += jnp.dot(hn, wb_ref[...], preferred_element_type=_f32)
    z = jnp.maximum(z, 0.0)
    n = jnp.sqrt(jnp.sum(z * z, axis=1, keepdims=True))
    zn = z / jnp.maximum(n, 1e-12)
    h1f_ref[...] = zn.astype(_bf16)
    h1_ref[...] = jnp.dot(zn.astype(_bf16), wn_ref[...],
                          preferred_element_type=_f32).astype(_bf16)


def _final_kernel(h1f_ref, agg_ref, deg_ref, wa_ref, wb_ref, wc_ref, b_ref,
                  out_ref):
    hn = (agg_ref[...] / deg_ref[...]).astype(_bf16)
    z = jnp.dot(h1f_ref[...], wa_ref[...], preferred_element_type=_f32)
    z += jnp.dot(hn, wb_ref[...], preferred_element_type=_f32)
    n = jnp.sqrt(jnp.sum(z * z, axis=1, keepdims=True))
    zn = (z / jnp.maximum(n, 1e-12)).astype(_bf16)
    out_ref[...] = jnp.dot(zn, wc_ref[...],
                           preferred_element_type=_f32) + b_ref[...]


def _row_spec(bm, cols):
    return pl.BlockSpec((bm, cols), lambda i: (i, 0))


def _full_spec(rows, cols):
    return pl.BlockSpec((rows, cols), lambda i: (0, 0))


def _forward_shard(adj_l, x_full, wn0, wl0a, wl0b, wn1, wl1a, wl1b, wc, bc):
    nl = adj_l.shape[0]
    grid_big = (nl // BM_BIG,)
    grid_small_l = (nl // BM_SMALL,)
    grid_small_full = (N // BM_SMALL,)

    h0 = pl.pallas_call(
        _neigh_lin_kernel,
        grid=grid_small_full,
        in_specs=[_row_spec(BM_SMALL, D), _full_spec(D, D)],
        out_specs=_row_spec(BM_SMALL, D),
        out_shape=jax.ShapeDtypeStruct((N, D), _bf16),
    )(x_full, wn0)

    agg0, deg = pl.pallas_call(
        _agg_deg_kernel,
        grid=grid_big,
        in_specs=[_row_spec(BM_BIG, N), _full_spec(N, D)],
        out_specs=[_row_spec(BM_BIG, D), _row_spec(BM_BIG, 1)],
        out_shape=[jax.ShapeDtypeStruct((nl, D), _f32),
                   jax.ShapeDtypeStruct((nl, 1), _f32)],
    )(adj_l, h0)

    row0 = jax.lax.axis_index("x") * nl
    x_l = jax.lax.dynamic_slice_in_dim(x_full, row0, nl, axis=0)

    h1f, h1_l = pl.pallas_call(
        _mid_kernel,
        grid=grid_small_l,
        in_specs=[_row_spec(BM_SMALL, D), _row_spec(BM_SMALL, D),
                  _row_spec(BM_SMALL, 1), _full_spec(D, H),
                  _full_spec(D, H), _full_spec(H, H)],
        out_specs=[_row_spec(BM_SMALL, H), _row_spec(BM_SMALL, H)],
        out_shape=[jax.ShapeDtypeStruct((nl, H), _bf16),
                   jax.ShapeDtypeStruct((nl, H), _bf16)],
    )(x_l, agg0, deg, wl0a, wl0b, wn1)

    h1 = jax.lax.all_gather(h1_l, "x", axis=0, tiled=True)

    agg1 = pl.pallas_call(
        _agg_kernel,
        grid=grid_big,
        in_specs=[_row_spec(BM_BIG, N), _full_spec(N, H)],
        out_specs=_row_spec(BM_BIG, H),
        out_shape=jax.ShapeDtypeStruct((nl, H), _f32),
    )(adj_l, h1)

    out_l = pl.pallas_call(
        _final_kernel,
        grid=grid_small_l,
        in_specs=[_row_spec(BM_SMALL, H), _row_spec(BM_SMALL, H),
                  _row_spec(BM_SMALL, 1), _full_spec(H, H),
                  _full_spec(H, H), _full_spec(H, C), _full_spec(1, C)],
        out_specs=_row_spec(BM_SMALL, C),
        out_shape=jax.ShapeDtypeStruct((nl, C), _f32),
    )(h1f, agg1, deg, wl1a, wl1b, wc, bc)

    return out_l


def kernel(adj, inputs, neigh_feats, W_neigh0, W_lin0, W_neigh1, W_lin1,
           W_clf, b_clf):
    del neigh_feats

    wn0 = W_neigh0.T.astype(_bf16)
    wl0a = W_lin0[:, :D].T.astype(_bf16)
    wl0b = W_lin0[:, D:].T.astype(_bf16)
    wn1 = W_neigh1.T.astype(_bf16)
    wl1a = W_lin1[:, :H].T.astype(_bf16)
    wl1b = W_lin1[:, H:].T.astype(_bf16)
    wc = W_clf.T.astype(_bf16)
    bc = b_clf.reshape(1, C)

    devs = jax.devices()
    ndev = 2 if len(devs) >= 2 else 1
    mesh = Mesh(np.array(devs[:ndev]), ("x",))
    rep = P(None, None)
    fwd = shard_map(
        _forward_shard,
        mesh=mesh,
        in_specs=(P("x", None), rep, rep, rep, rep, rep, rep, rep, rep, rep),
        out_specs=P("x", None),
        check_rep=False,
    )
    return fwd(adj, inputs, wn0, wl0a, wl0b, wn1, wl1a, wl1b, wc, bc)


# speedup vs baseline: 3.3096x; 3.3096x over previous
import jax
import jax.numpy as jnp
from jax.experimental import pallas as pl

N = 8192
D = 256
H = 256
C = 64

BM_BIG = 512
BM_SMALL = 1024

_bf16 = jnp.bfloat16
_f32 = jnp.float32


def _neigh_lin_kernel(x_ref, w_ref, o_ref):
    o_ref[...] = jnp.dot(
        x_ref[...].astype(_bf16), w_ref[...], preferred_element_type=_f32
    ).astype(_bf16)


def _agg_deg_kernel(adj_ref, h_ref, agg_ref, deg_ref):
    a = adj_ref[...]
    deg_ref[...] = jnp.sum(a, axis=1, keepdims=True) + 1.0
    agg_ref[...] = jnp.dot(
        a.astype(_bf16), h_ref[...], preferred_element_type=_f32
    )


def _agg_kernel(adj_ref, h_ref, agg_ref):
    agg_ref[...] = jnp.dot(
        adj_ref[...].astype(_bf16), h_ref[...], preferred_element_type=_f32
    )


def _mid_kernel(x_ref, agg_ref, deg_ref, wa_ref, wb_ref, wn_ref,
                h1f_ref, h1_ref):
    hn = (agg_ref[...] / deg_ref[...]).astype(_bf16)
    z = jnp.dot(x_ref[...].astype(_bf16), wa_ref[...],
                preferred_element_type=_f32)
    z += jnp.dot(hn, wb_ref[...], preferred_element_type=_f32)
    z = jnp.maximum(z, 0.0)
    n = jnp.sqrt(jnp.sum(z * z, axis=1, keepdims=True))
    zn = z / jnp.maximum(n, 1e-12)
    h1f_ref[...] = zn.astype(_bf16)
    h1_ref[...] = jnp.dot(zn.astype(_bf16), wn_ref[...],
                          preferred_element_type=_f32).astype(_bf16)


def _final_kernel(h1f_ref, agg_ref, deg_ref, wa_ref, wb_ref, wc_ref, b_ref,
                  out_ref):
    hn = (agg_ref[...] / deg_ref[...]).astype(_bf16)
    z = jnp.dot(h1f_ref[...], wa_ref[...], preferred_element_type=_f32)
    z += jnp.dot(hn, wb_ref[...], preferred_element_type=_f32)
    n = jnp.sqrt(jnp.sum(z * z, axis=1, keepdims=True))
    zn = (z / jnp.maximum(n, 1e-12)).astype(_bf16)
    out_ref[...] = jnp.dot(zn, wc_ref[...],
                           preferred_element_type=_f32) + b_ref[...]


def _row_spec(bm, cols):
    return pl.BlockSpec((bm, cols), lambda i: (i, 0))


def _full_spec(rows, cols):
    return pl.BlockSpec((rows, cols), lambda i: (0, 0))


def kernel(adj, inputs, neigh_feats, W_neigh0, W_lin0, W_neigh1, W_lin1,
           W_clf, b_clf):
    del neigh_feats
    grid_big = (N // BM_BIG,)
    grid_small = (N // BM_SMALL,)

    wn0 = W_neigh0.T.astype(_bf16)
    wl0a = W_lin0[:, :D].T.astype(_bf16)
    wl0b = W_lin0[:, D:].T.astype(_bf16)
    wn1 = W_neigh1.T.astype(_bf16)
    wl1a = W_lin1[:, :H].T.astype(_bf16)
    wl1b = W_lin1[:, H:].T.astype(_bf16)
    wc = W_clf.T.astype(_bf16)
    bc = b_clf.reshape(1, C)

    h0 = pl.pallas_call(
        _neigh_lin_kernel,
        grid=grid_small,
        in_specs=[_row_spec(BM_SMALL, D), _full_spec(D, D)],
        out_specs=_row_spec(BM_SMALL, D),
        out_shape=jax.ShapeDtypeStruct((N, D), _bf16),
    )(inputs, wn0)

    agg0, deg = pl.pallas_call(
        _agg_deg_kernel,
        grid=grid_big,
        in_specs=[_row_spec(BM_BIG, N), _full_spec(N, D)],
        out_specs=[_row_spec(BM_BIG, D), _row_spec(BM_BIG, 1)],
        out_shape=[jax.ShapeDtypeStruct((N, D), _f32),
                   jax.ShapeDtypeStruct((N, 1), _f32)],
    )(adj, h0)

    h1f, h1 = pl.pallas_call(
        _mid_kernel,
        grid=grid_small,
        in_specs=[_row_spec(BM_SMALL, D), _row_spec(BM_SMALL, D),
                  _row_spec(BM_SMALL, 1), _full_spec(D, H),
                  _full_spec(D, H), _full_spec(H, H)],
        out_specs=[_row_spec(BM_SMALL, H), _row_spec(BM_SMALL, H)],
        out_shape=[jax.ShapeDtypeStruct((N, H), _bf16),
                   jax.ShapeDtypeStruct((N, H), _bf16)],
    )(inputs, agg0, deg, wl0a, wl0b, wn1)

    agg1 = pl.pallas_call(
        _agg_kernel,
        grid=grid_big,
        in_specs=[_row_spec(BM_BIG, N), _full_spec(N, H)],
        out_specs=_row_spec(BM_BIG, H),
        out_shape=jax.ShapeDtypeStruct((N, H), _f32),
    )(adj, h1)

    out = pl.pallas_call(
        _final_kernel,
        grid=grid_small,
        in_specs=[_row_spec(BM_SMALL, H), _row_spec(BM_SMALL, H),
                  _row_spec(BM_SMALL, 1), _full_spec(H, H),
                  _full_spec(H, H), _full_spec(H, C), _full_spec(1, C)],
        out_specs=_row_spec(BM_SMALL, C),
        out_shape=jax.ShapeDtypeStruct((N, C), _f32),
    )(h1f, agg1, deg, wl1a, wl1b, wc, bc)

    return out


# speedup vs baseline: 3.4816x; 1.0520x over previous
import jax
import jax.numpy as jnp
from jax.experimental import pallas as pl
from jax.experimental.pallas import tpu as pltpu

N = 8192
D = 256
H = 256
C = 64

BM = 256

_bf16 = jnp.bfloat16
_f32 = jnp.float32


def _pass_a_kernel(x_full_ref, adj_ref, x_ref, wn0_ref, wa_ref, wb_ref,
                   wn1_ref, h1f_ref, h1_ref, deg_ref, h0_scr):
    @pl.when(pl.program_id(0) == 0)
    def _():
        h0_scr[...] = jnp.dot(
            x_full_ref[...].astype(_bf16), wn0_ref[...],
            preferred_element_type=_f32,
        ).astype(_bf16)

    a = adj_ref[...]
    deg = jnp.sum(a, axis=1, keepdims=True) + 1.0
    deg_ref[...] = deg
    agg = jnp.dot(a.astype(_bf16), h0_scr[...], preferred_element_type=_f32)
    hn = (agg / deg).astype(_bf16)
    z = jnp.dot(x_ref[...].astype(_bf16), wa_ref[...],
                preferred_element_type=_f32)
    z += jnp.dot(hn, wb_ref[...], preferred_element_type=_f32)
    z = jnp.maximum(z, 0.0)
    n = jnp.sqrt(jnp.sum(z * z, axis=1, keepdims=True))
    zn = z / jnp.maximum(n, 1e-12)
    h1f_ref[...] = zn.astype(_bf16)
    h1_ref[...] = jnp.dot(zn.astype(_bf16), wn1_ref[...],
                          preferred_element_type=_f32).astype(_bf16)


def _pass_b_kernel(adj_ref, h1_ref, h1f_ref, deg_ref, wa_ref, wb_ref, wc_ref,
                   b_ref, out_ref):
    agg = jnp.dot(adj_ref[...].astype(_bf16), h1_ref[...],
                  preferred_element_type=_f32)
    hn = (agg / deg_ref[...]).astype(_bf16)
    z = jnp.dot(h1f_ref[...], wa_ref[...], preferred_element_type=_f32)
    z += jnp.dot(hn, wb_ref[...], preferred_element_type=_f32)
    n = jnp.sqrt(jnp.sum(z * z, axis=1, keepdims=True))
    zn = (z / jnp.maximum(n, 1e-12)).astype(_bf16)
    out_ref[...] = jnp.dot(zn, wc_ref[...],
                           preferred_element_type=_f32) + b_ref[...]


def _row_spec(bm, cols):
    return pl.BlockSpec((bm, cols), lambda i: (i, 0))


def _full_spec(rows, cols):
    return pl.BlockSpec((rows, cols), lambda i: (0, 0))


def kernel(adj, inputs, neigh_feats, W_neigh0, W_lin0, W_neigh1, W_lin1,
           W_clf, b_clf):
    del neigh_feats
    grid = (N // BM,)

    wn0 = W_neigh0.T.astype(_bf16)
    wl0a = W_lin0[:, :D].T.astype(_bf16)
    wl0b = W_lin0[:, D:].T.astype(_bf16)
    wn1 = W_neigh1.T.astype(_bf16)
    wl1a = W_lin1[:, :H].T.astype(_bf16)
    wl1b = W_lin1[:, H:].T.astype(_bf16)
    wc = W_clf.T.astype(_bf16)
    bc = b_clf.reshape(1, C)

    h1f, h1, deg = pl.pallas_call(
        _pass_a_kernel,
        grid=grid,
        in_specs=[_full_spec(N, D), _row_spec(BM, N), _row_spec(BM, D),
                  _full_spec(D, D), _full_spec(D, H), _full_spec(D, H),
                  _full_spec(H, H)],
        out_specs=[_row_spec(BM, H), _row_spec(BM, H), _row_spec(BM, 1)],
        out_shape=[jax.ShapeDtypeStruct((N, H), _bf16),
                   jax.ShapeDtypeStruct((N, H), _bf16),
                   jax.ShapeDtypeStruct((N, 1), _f32)],
        scratch_shapes=[pltpu.VMEM((N, D), _bf16)],
    )(inputs, adj, inputs, wn0, wl0a, wl0b, wn1)

    out = pl.pallas_call(
        _pass_b_kernel,
        grid=grid,
        in_specs=[_row_spec(BM, N), _full_spec(N, H), _row_spec(BM, H),
                  _row_spec(BM, 1), _full_spec(H, H), _full_spec(H, H),
                  _full_spec(H, C), _full_spec(1, C)],
        out_specs=_row_spec(BM, C),
        out_shape=jax.ShapeDtypeStruct((N, C), _f32),
    )(adj, h1, h1f, deg, wl1a, wl1b, wc, bc)

    return out


# speedup vs baseline: 3.6301x; 1.0426x over previous
import jax
import jax.numpy as jnp
from jax.experimental import pallas as pl
from jax.experimental.pallas import tpu as pltpu

N = 8192
D = 256
H = 256
C = 64

BM = 256

_bf16 = jnp.bfloat16
_f32 = jnp.float32


def _dot_t(x, w):
    return jax.lax.dot_general(
        x.astype(_bf16), w.astype(_bf16),
        (((1,), (1,)), ((), ())), preferred_element_type=_f32)


def _pass_a_kernel(x_full_ref, adj_ref, x_ref, wn0_ref, wl0_ref, wn1_ref,
                   h1f_ref, h1_ref, deg_ref, h0_scr):
    @pl.when(pl.program_id(0) == 0)
    def _():
        h0_scr[...] = _dot_t(x_full_ref[...], wn0_ref[...]).astype(_bf16)

    a = adj_ref[...]
    deg = jnp.sum(a, axis=1, keepdims=True) + 1.0
    deg_ref[...] = deg
    agg = jnp.dot(a.astype(_bf16), h0_scr[...], preferred_element_type=_f32)
    hn = (agg / deg).astype(_bf16)
    wl0 = wl0_ref[...]
    z = _dot_t(x_ref[...], wl0[:, :D]) + _dot_t(hn, wl0[:, D:])
    z = jnp.maximum(z, 0.0)
    n = jnp.sqrt(jnp.sum(z * z, axis=1, keepdims=True))
    zn = z / jnp.maximum(n, 1e-12)
    h1f_ref[...] = zn.astype(_bf16)
    h1_ref[...] = _dot_t(zn, wn1_ref[...]).astype(_bf16)


def _pass_b_kernel(adj_ref, h1_ref, h1f_ref, deg_ref, wl1_ref, wc_ref,
                   b_ref, out_ref):
    agg = jnp.dot(adj_ref[...].astype(_bf16), h1_ref[...],
                  preferred_element_type=_f32)
    hn = (agg / deg_ref[...]).astype(_bf16)
    wl1 = wl1_ref[...]
    z = _dot_t(h1f_ref[...], wl1[:, :H]) + _dot_t(hn, wl1[:, H:])
    n = jnp.sqrt(jnp.sum(z * z, axis=1, keepdims=True))
    zn = z / jnp.maximum(n, 1e-12)
    out_ref[...] = _dot_t(zn, wc_ref[...]) + b_ref[...]


def _row_spec(bm, cols):
    return pl.BlockSpec((bm, cols), lambda i: (i, 0))


def _full_spec(rows, cols):
    return pl.BlockSpec((rows, cols), lambda i: (0, 0))


def kernel(adj, inputs, neigh_feats, W_neigh0, W_lin0, W_neigh1, W_lin1,
           W_clf, b_clf):
    del neigh_feats
    grid = (N // BM,)
    bc = b_clf.reshape(1, C)

    h1f, h1, deg = pl.pallas_call(
        _pass_a_kernel,
        grid=grid,
        in_specs=[_full_spec(N, D), _row_spec(BM, N), _row_spec(BM, D),
                  _full_spec(D, D), _full_spec(H, 2 * D), _full_spec(H, H)],
        out_specs=[_row_spec(BM, H), _row_spec(BM, H), _row_spec(BM, 1)],
        out_shape=[jax.ShapeDtypeStruct((N, H), _bf16),
                   jax.ShapeDtypeStruct((N, H), _bf16),
                   jax.ShapeDtypeStruct((N, 1), _f32)],
        scratch_shapes=[pltpu.VMEM((N, D), _bf16)],
    )(inputs, adj, inputs, W_neigh0, W_lin0, W_neigh1)

    out = pl.pallas_call(
        _pass_b_kernel,
        grid=grid,
        in_specs=[_row_spec(BM, N), _full_spec(N, H), _row_spec(BM, H),
                  _row_spec(BM, 1), _full_spec(H, 2 * H), _full_spec(C, H),
                  _full_spec(1, C)],
        out_specs=_row_spec(BM, C),
        out_shape=jax.ShapeDtypeStruct((N, C), _f32),
    )(adj, h1, h1f, deg, W_lin1, W_clf, bc)

    return out


# speedup vs baseline: 3.7772x; 1.0405x over previous
import jax
import jax.numpy as jnp
from jax.experimental import pallas as pl
from jax.experimental.pallas import tpu as pltpu

N = 8192
D = 256
H = 256
C = 64

BM = 512

_bf16 = jnp.bfloat16
_f32 = jnp.float32


def _dot_t(x, w):
    return jax.lax.dot_general(
        x.astype(_bf16), w.astype(_bf16),
        (((1,), (1,)), ((), ())), preferred_element_type=_f32)


def _pass_a_kernel(x_full_ref, adj_ref, x_ref, wn0_ref, wl0_ref, wn1_ref,
                   h1f_ref, h1_ref, deg_ref, h0_scr):
    @pl.when(pl.program_id(0) == 0)
    def _():
        h0_scr[...] = _dot_t(x_full_ref[...], wn0_ref[...]).astype(_bf16)

    a = adj_ref[...]
    deg = jnp.sum(a, axis=1, keepdims=True) + 1.0
    deg_ref[...] = deg
    agg = jnp.dot(a.astype(_bf16), h0_scr[...], preferred_element_type=_f32)
    hn = (agg / deg).astype(_bf16)
    wl0 = wl0_ref[...]
    z = _dot_t(x_ref[...], wl0[:, :D]) + _dot_t(hn, wl0[:, D:])
    z = jnp.maximum(z, 0.0)
    n = jnp.sqrt(jnp.sum(z * z, axis=1, keepdims=True))
    zn = z / jnp.maximum(n, 1e-12)
    h1f_ref[...] = zn.astype(_bf16)
    h1_ref[...] = _dot_t(zn, wn1_ref[...]).astype(_bf16)


def _pass_b_kernel(adj_ref, h1_ref, h1f_ref, deg_ref, wl1_ref, wc_ref,
                   b_ref, out_ref):
    agg = jnp.dot(adj_ref[...].astype(_bf16), h1_ref[...],
                  preferred_element_type=_f32)
    hn = (agg / deg_ref[...]).astype(_bf16)
    wl1 = wl1_ref[...]
    z = _dot_t(h1f_ref[...], wl1[:, :H]) + _dot_t(hn, wl1[:, H:])
    n = jnp.sqrt(jnp.sum(z * z, axis=1, keepdims=True))
    zn = z / jnp.maximum(n, 1e-12)
    out_ref[...] = _dot_t(zn, wc_ref[...]) + b_ref[...]


def _row_spec(bm, cols):
    return pl.BlockSpec((bm, cols), lambda i: (i, 0))


def _full_spec(rows, cols):
    return pl.BlockSpec((rows, cols), lambda i: (0, 0))


def kernel(adj, inputs, neigh_feats, W_neigh0, W_lin0, W_neigh1, W_lin1,
           W_clf, b_clf):
    del neigh_feats
    grid = (N // BM,)
    bc = b_clf.reshape(1, C)

    h1f, h1, deg = pl.pallas_call(
        _pass_a_kernel,
        grid=grid,
        in_specs=[_full_spec(N, D), _row_spec(BM, N), _row_spec(BM, D),
                  _full_spec(D, D), _full_spec(H, 2 * D), _full_spec(H, H)],
        out_specs=[_row_spec(BM, H), _row_spec(BM, H), _row_spec(BM, 1)],
        out_shape=[jax.ShapeDtypeStruct((N, H), _bf16),
                   jax.ShapeDtypeStruct((N, H), _bf16),
                   jax.ShapeDtypeStruct((N, 1), _f32)],
        scratch_shapes=[pltpu.VMEM((N, D), _bf16)],
    )(inputs, adj, inputs, W_neigh0, W_lin0, W_neigh1)

    out = pl.pallas_call(
        _pass_b_kernel,
        grid=grid,
        in_specs=[_row_spec(BM, N), _full_spec(N, H), _row_spec(BM, H),
                  _row_spec(BM, 1), _full_spec(H, 2 * H), _full_spec(C, H),
                  _full_spec(1, C)],
        out_specs=_row_spec(BM, C),
        out_shape=jax.ShapeDtypeStruct((N, C), _f32),
    )(adj, h1, h1f, deg, W_lin1, W_clf, bc)

    return out
